# baseline (speedup 1.0000x reference)
import functools

import numpy as np

import jax
import jax.numpy as jnp
from jax import lax
from jax.experimental import pallas as pl
from jax.experimental.pallas import tpu as pltpu
from jax.experimental.pallas import tpu_sc as plsc

N = 10000
E = 320000
D = 128

NC, NS = 2, 16
TILES = NC * NS
NPAD = 10240
CW = 128
EPT = NPAD
CHUNKS = EPT // CW
E_PAD = TILES * EPT
STRIPE = NPAD // NS
BLK = 1024


def _mesh():
    return plsc.VectorSubcoreMesh(
        core_axis_name="c", subcore_axis_name="s", num_cores=NC, num_subcores=NS
    )


def _deg_body(dst_r, zeros, ones, out, idx_v, obuf, deg_sh,
              semd0, semd1, semd2, semd3):
    c = lax.axis_index("c")
    s = lax.axis_index("s")
    wid = s * NC + c
    stripe = s * STRIPE
    pltpu.sync_copy(zeros.at[pl.ds(stripe, STRIPE)],
                    deg_sh.at[pl.ds(stripe, STRIPE)])
    pltpu.sync_copy(ones, obuf)
    pltpu.sync_copy(dst_r.at[wid], idx_v)
    plsc.subcore_barrier()

    semd = (semd0, semd1, semd2, semd3)

    def body(jj, carry):
        for b in range(4):
            j = jj * 4 + b

            @pl.when(j >= 4)
            def _():
                pltpu.make_async_copy(
                    obuf, deg_sh.at[idx_v.at[j]], semd[b]
                ).wait()

            pltpu.async_copy(obuf, deg_sh.at[idx_v.at[j]], semd[b], add=True)
        return carry

    lax.fori_loop(0, CHUNKS // 4, body, 0)
    for b in range(4):
        pltpu.make_async_copy(obuf, deg_sh.at[idx_v.at[0]], semd[b]).wait()
    plsc.subcore_barrier()
    pltpu.sync_copy(deg_sh.at[pl.ds(stripe, STRIPE)],
                    out.at[c, pl.ds(stripe, STRIPE)])


@functools.cache
def _sc_degree_kernel():
    return pl.kernel(
        _deg_body,
        out_type=jax.ShapeDtypeStruct((NC, NPAD, D), jnp.float32),
        mesh=_mesh(),
        scratch_types=[
            pltpu.VMEM((CHUNKS, CW), jnp.int32),
            pltpu.VMEM((CW, D), jnp.float32),
            pltpu.VMEM_SHARED((NPAD, D), jnp.float32),
            pltpu.SemaphoreType.DMA,
            pltpu.SemaphoreType.DMA,
            pltpu.SemaphoreType.DMA,
            pltpu.SemaphoreType.DMA,
        ],
    )


G = 8
NGRP = CHUNKS // G


def _agg_body(g, src_r, dst_r, zeros, part, sr, dst_v, gbuf, acc_sh,
              semg0, semg1, sems0, sems1):
    c = lax.axis_index("c")
    s = lax.axis_index("s")
    wid = s * NC + c
    stripe = s * STRIPE
    pltpu.sync_copy(zeros.at[pl.ds(stripe, STRIPE)],
                    acc_sh.at[pl.ds(stripe, STRIPE)])
    pltpu.sync_copy(dst_r.at[wid], dst_v)
    pltpu.sync_copy(src_r.at[wid, pl.ds(0, G)], sr.at[0])
    pltpu.sync_copy(src_r.at[wid, pl.ds(G, G)], sr.at[1])
    plsc.subcore_barrier()

    semg = (semg0, semg1)
    sems = (sems0, sems1)
    pltpu.async_copy(g.at[sr.at[0, 0]], gbuf.at[0], semg0)

    def body(jj, carry):
        for b in range(2):
            bb = 1 - b
            j = jj * 2 + b

            @pl.when(j >= 1)
            def _():
                pltpu.make_async_copy(
                    gbuf.at[bb], acc_sh.at[dst_v.at[j]], sems[bb]
                ).wait()

            @pl.when(j + 1 < CHUNKS)
            def _():
                jn = j + 1
                pltpu.async_copy(
                    g.at[sr.at[(jn // G) % 2, jn % G]], gbuf.at[bb], semg[bb]
                )

            pltpu.make_async_copy(
                g.at[sr.at[0, 0]], gbuf.at[b], semg[b]
            ).wait()
            pltpu.async_copy(gbuf.at[b], acc_sh.at[dst_v.at[j]], sems[b],
                             add=True)

            @pl.when(jnp.logical_and(j % G == G - 1, j // G + 2 < NGRP))
            def _():
                grp = j // G
                pltpu.sync_copy(
                    src_r.at[wid, pl.ds((grp + 2) * G, G)], sr.at[grp % 2]
                )
        return carry

    lax.fori_loop(0, CHUNKS // 2, body, 0)
    pltpu.make_async_copy(gbuf.at[1], acc_sh.at[dst_v.at[0]], sems1).wait()
    plsc.subcore_barrier()
    pltpu.sync_copy(acc_sh.at[pl.ds(stripe, STRIPE)],
                    part.at[c, pl.ds(stripe, STRIPE)])


@functools.cache
def _sc_agg_kernel():
    return pl.kernel(
        _agg_body,
        out_type=jax.ShapeDtypeStruct((NC, NPAD, D), jnp.float32),
        mesh=_mesh(),
        scratch_types=[
            pltpu.VMEM((2, G, CW), jnp.int32),
            pltpu.VMEM((CHUNKS, CW), jnp.int32),
            pltpu.VMEM((2, CW, D), jnp.float32),
            pltpu.VMEM_SHARED((NPAD, D), jnp.float32),
            pltpu.SemaphoreType.DMA,
            pltpu.SemaphoreType.DMA,
            pltpu.SemaphoreType.DMA,
            pltpu.SemaphoreType.DMA,
        ],
    )


BLKE = E_PAD // 5


def _prep_body(ei_ref, out_ref):
    i = pl.program_id(0)
    col = jax.lax.broadcasted_iota(jnp.int32, (1, BLKE), 1) + i * BLKE
    pidx = col - E
    ps = (pidx * 37) % N
    pd = N + pidx % (NPAD - N)
    out_ref[...] = jnp.where(
        col < E, ei_ref[...], jnp.concatenate([ps, pd], axis=0)
    )


def _tc_prep(edge_index):
    return pl.pallas_call(
        _prep_body,
        grid=(5,),
        in_specs=[pl.BlockSpec((2, BLKE), lambda i: (0, i))],
        out_specs=pl.BlockSpec((2, BLKE), lambda i: (0, i)),
        out_shape=jax.ShapeDtypeStruct((2, E_PAD), jnp.int32),
    )(edge_index)


def _tc_mm_body(x_ref, w_ref, h_ref):
    h_ref[...] = jnp.dot(
        x_ref[...], w_ref[...], preferred_element_type=jnp.float32
    )


def _tc_scale_body(h_ref, deg_ref, g_ref, dis_ref):
    deg = deg_ref[0][:, 0:1] + deg_ref[1][:, 0:1] + 1.0
    dis = lax.rsqrt(deg)
    g_ref[...] = h_ref[...] * dis
    dis_ref[...] = dis


def _tc_mid_body(p_ref, g1_ref, dis_ref, b1_ref, w2_ref, g2_ref):
    dis = dis_ref[...]
    a = (p_ref[0] + p_ref[1] + g1_ref[...]) * dis + b1_ref[...]
    h = jnp.maximum(a, 0.0)
    g2_ref[...] = (
        jnp.dot(h, w2_ref[...], preferred_element_type=jnp.float32) * dis
    )


def _tc_out_body(p_ref, g2_ref, dis_ref, b2_ref, o_ref):
    z = (p_ref[0] + p_ref[1] + g2_ref[...]) * dis_ref[...] + b2_ref[...]
    m = jnp.max(z, axis=1, keepdims=True)
    sh = z - m
    o_ref[...] = sh - jnp.log(jnp.sum(jnp.exp(sh), axis=1, keepdims=True))


_GRID = (NPAD // BLK,)


def _tc_mm(x_p, W1):
    return pl.pallas_call(
        _tc_mm_body,
        grid=_GRID,
        in_specs=[
            pl.BlockSpec((BLK, D), lambda i: (i, 0)),
            pl.BlockSpec((D, D), lambda i: (0, 0)),
        ],
        out_specs=pl.BlockSpec((BLK, D), lambda i: (i, 0)),
        out_shape=jax.ShapeDtypeStruct((NPAD, D), jnp.float32),
    )(x_p, W1)


def _tc_scale(h1, degp):
    return pl.pallas_call(
        _tc_scale_body,
        grid=_GRID,
        in_specs=[
            pl.BlockSpec((BLK, D), lambda i: (i, 0)),
            pl.BlockSpec((NC, BLK, D), lambda i: (0, i, 0)),
        ],
        out_specs=[
            pl.BlockSpec((BLK, D), lambda i: (i, 0)),
            pl.BlockSpec((BLK, 1), lambda i: (i, 0)),
        ],
        out_shape=[
            jax.ShapeDtypeStruct((NPAD, D), jnp.float32),
            jax.ShapeDtypeStruct((NPAD, 1), jnp.float32),
        ],
    )(h1, degp)


def _tc_mid(part1, g1, dis, b1r, W2):
    return pl.pallas_call(
        _tc_mid_body,
        grid=_GRID,
        in_specs=[
            pl.BlockSpec((NC, BLK, D), lambda i: (0, i, 0)),
            pl.BlockSpec((BLK, D), lambda i: (i, 0)),
            pl.BlockSpec((BLK, 1), lambda i: (i, 0)),
            pl.BlockSpec((1, D), lambda i: (0, 0)),
            pl.BlockSpec((D, D), lambda i: (0, 0)),
        ],
        out_specs=pl.BlockSpec((BLK, D), lambda i: (i, 0)),
        out_shape=jax.ShapeDtypeStruct((NPAD, D), jnp.float32),
    )(part1, g1, dis, b1r, W2)


def _tc_out(part2, g2, dis, b2r):
    return pl.pallas_call(
        _tc_out_body,
        grid=_GRID,
        in_specs=[
            pl.BlockSpec((NC, BLK, D), lambda i: (0, i, 0)),
            pl.BlockSpec((BLK, D), lambda i: (i, 0)),
            pl.BlockSpec((BLK, 1), lambda i: (i, 0)),
            pl.BlockSpec((1, D), lambda i: (0, 0)),
        ],
        out_specs=pl.BlockSpec((BLK, D), lambda i: (i, 0)),
        out_shape=jax.ShapeDtypeStruct((N, D), jnp.float32),
    )(part2, g2, dis, b2r)


def kernel(x, edge_index, W1, b1, W2, b2):
    ep = _tc_prep(edge_index)
    src_p = ep[0].reshape(TILES, CHUNKS, CW)
    dst_p = ep[1].reshape(TILES, CHUNKS, CW)
    x_p = jnp.concatenate([x, jnp.zeros((NPAD - N, D), x.dtype)], axis=0)
    zeros128 = jnp.zeros((NPAD, D), jnp.float32)
    ones128 = jnp.ones((CW, D), jnp.float32)
    b1r = b1.reshape(1, D)
    b2r = b2.reshape(1, D)

    degp = _sc_degree_kernel()(dst_p, zeros128, ones128)
    h1 = _tc_mm(x_p, W1)
    g1, dis = _tc_scale(h1, degp)
    part1 = _sc_agg_kernel()(g1, src_p, dst_p, zeros128)
    g2 = _tc_mid(part1, g1, dis, b1r, W2)
    part2 = _sc_agg_kernel()(g2, src_p, dst_p, zeros128)
    return _tc_out(part2, g2, dis, b2r)


# speedup vs baseline: 1.0187x; 1.0187x over previous
import functools

import numpy as np

import jax
import jax.numpy as jnp
from jax import lax
from jax.experimental import pallas as pl
from jax.experimental.pallas import tpu as pltpu
from jax.experimental.pallas import tpu_sc as plsc

N = 10000
E = 320000
D = 128

NC, NS = 2, 16
TILES = NC * NS
NPAD = 10240
CW = 128
EPT = NPAD
CHUNKS = EPT // CW
E_PAD = TILES * EPT
STRIPE = NPAD // NS
BLK = 1024


def _mesh():
    return plsc.VectorSubcoreMesh(
        core_axis_name="c", subcore_axis_name="s", num_cores=NC, num_subcores=NS
    )


def _deg_body(dst_r, zeros, ones, out, idx_v, obuf, deg_sh,
              semd0, semd1, semd2, semd3):
    c = lax.axis_index("c")
    s = lax.axis_index("s")
    wid = s * NC + c
    stripe = s * STRIPE
    pltpu.sync_copy(zeros.at[pl.ds(stripe, STRIPE)],
                    deg_sh.at[pl.ds(stripe, STRIPE)])
    pltpu.sync_copy(ones, obuf)
    pltpu.sync_copy(dst_r.at[wid], idx_v)
    plsc.subcore_barrier()

    semd = (semd0, semd1, semd2, semd3)

    def body(jj, carry):
        for b in range(4):
            j = jj * 4 + b

            @pl.when(j >= 4)
            def _():
                pltpu.make_async_copy(
                    obuf, deg_sh.at[idx_v.at[j]], semd[b]
                ).wait()

            pltpu.async_copy(obuf, deg_sh.at[idx_v.at[j]], semd[b], add=True)
        return carry

    lax.fori_loop(0, CHUNKS // 4, body, 0)
    for b in range(4):
        pltpu.make_async_copy(obuf, deg_sh.at[idx_v.at[0]], semd[b]).wait()
    plsc.subcore_barrier()
    pltpu.sync_copy(deg_sh.at[pl.ds(stripe, STRIPE)],
                    out.at[c, pl.ds(stripe, STRIPE)])


@functools.cache
def _sc_degree_kernel():
    return pl.kernel(
        _deg_body,
        out_type=jax.ShapeDtypeStruct((NC, NPAD, D), jnp.float32),
        mesh=_mesh(),
        scratch_types=[
            pltpu.VMEM((CHUNKS, CW), jnp.int32),
            pltpu.VMEM((CW, D), jnp.float32),
            pltpu.VMEM_SHARED((NPAD, D), jnp.float32),
            pltpu.SemaphoreType.DMA,
            pltpu.SemaphoreType.DMA,
            pltpu.SemaphoreType.DMA,
            pltpu.SemaphoreType.DMA,
        ],
    )


G = 8
NGRP = CHUNKS // G


def _agg_body(g, src_r, dst_r, zeros, part, sr, dst_v, gbuf, acc_sh,
              semg0, semg1, sems0, sems1):
    c = lax.axis_index("c")
    s = lax.axis_index("s")
    wid = s * NC + c
    stripe = s * STRIPE
    pltpu.sync_copy(zeros.at[pl.ds(stripe, STRIPE)],
                    acc_sh.at[pl.ds(stripe, STRIPE)])
    pltpu.sync_copy(dst_r.at[wid], dst_v)
    pltpu.sync_copy(src_r.at[wid, pl.ds(0, G)], sr.at[0])
    pltpu.sync_copy(src_r.at[wid, pl.ds(G, G)], sr.at[1])
    plsc.subcore_barrier()

    semg = (semg0, semg1)
    sems = (sems0, sems1)
    pltpu.async_copy(g.at[sr.at[0, 0]], gbuf.at[0], semg0)

    def body(jj, carry):
        for b in range(2):
            bb = 1 - b
            j = jj * 2 + b

            @pl.when(j >= 1)
            def _():
                pltpu.make_async_copy(
                    gbuf.at[bb], acc_sh.at[dst_v.at[j]], sems[bb]
                ).wait()

            @pl.when(j + 1 < CHUNKS)
            def _():
                jn = j + 1
                pltpu.async_copy(
                    g.at[sr.at[(jn // G) % 2, jn % G]], gbuf.at[bb], semg[bb]
                )

            pltpu.make_async_copy(
                g.at[sr.at[0, 0]], gbuf.at[b], semg[b]
            ).wait()
            pltpu.async_copy(gbuf.at[b], acc_sh.at[dst_v.at[j]], sems[b],
                             add=True)

            @pl.when(jnp.logical_and(j % G == G - 1, j // G + 2 < NGRP))
            def _():
                grp = j // G
                pltpu.sync_copy(
                    src_r.at[wid, pl.ds((grp + 2) * G, G)], sr.at[grp % 2]
                )
        return carry

    lax.fori_loop(0, CHUNKS // 2, body, 0)
    pltpu.make_async_copy(gbuf.at[1], acc_sh.at[dst_v.at[0]], sems1).wait()
    plsc.subcore_barrier()
    pltpu.sync_copy(acc_sh.at[pl.ds(stripe, STRIPE)],
                    part.at[c, pl.ds(stripe, STRIPE)])


@functools.cache
def _sc_agg_kernel():
    return pl.kernel(
        _agg_body,
        out_type=jax.ShapeDtypeStruct((NC, NPAD, D), jnp.float32),
        mesh=_mesh(),
        scratch_types=[
            pltpu.VMEM((2, G, CW), jnp.int32),
            pltpu.VMEM((CHUNKS, CW), jnp.int32),
            pltpu.VMEM((2, CW, D), jnp.float32),
            pltpu.VMEM_SHARED((NPAD, D), jnp.float32),
            pltpu.SemaphoreType.DMA,
            pltpu.SemaphoreType.DMA,
            pltpu.SemaphoreType.DMA,
            pltpu.SemaphoreType.DMA,
        ],
    )


def _tc_mm_body(x_ref, w_ref, h_ref):
    h_ref[...] = jnp.dot(
        x_ref[...], w_ref[...], preferred_element_type=jnp.float32
    )


def _tc_scale_body(h_ref, deg_ref, g_ref, dis_ref):
    deg = deg_ref[0][:, 0:1] + deg_ref[1][:, 0:1] + 1.0
    dis = lax.rsqrt(deg)
    g_ref[...] = h_ref[...] * dis
    dis_ref[...] = dis


def _tc_mid_body(p_ref, g1_ref, dis_ref, b1_ref, w2_ref, g2_ref):
    dis = dis_ref[...]
    a = (p_ref[0] + p_ref[1] + g1_ref[...]) * dis + b1_ref[...]
    h = jnp.maximum(a, 0.0)
    g2_ref[...] = (
        jnp.dot(h, w2_ref[...], preferred_element_type=jnp.float32) * dis
    )


def _tc_out_body(p_ref, g2_ref, dis_ref, b2_ref, o_ref):
    z = (p_ref[0] + p_ref[1] + g2_ref[...]) * dis_ref[...] + b2_ref[...]
    m = jnp.max(z, axis=1, keepdims=True)
    sh = z - m
    o_ref[...] = sh - jnp.log(jnp.sum(jnp.exp(sh), axis=1, keepdims=True))


_GRID = (NPAD // BLK,)


def _tc_mm(x_p, W1):
    return pl.pallas_call(
        _tc_mm_body,
        grid=_GRID,
        in_specs=[
            pl.BlockSpec((BLK, D), lambda i: (i, 0)),
            pl.BlockSpec((D, D), lambda i: (0, 0)),
        ],
        out_specs=pl.BlockSpec((BLK, D), lambda i: (i, 0)),
        out_shape=jax.ShapeDtypeStruct((NPAD, D), jnp.float32),
    )(x_p, W1)


def _tc_scale(h1, degp):
    return pl.pallas_call(
        _tc_scale_body,
        grid=_GRID,
        in_specs=[
            pl.BlockSpec((BLK, D), lambda i: (i, 0)),
            pl.BlockSpec((NC, BLK, D), lambda i: (0, i, 0)),
        ],
        out_specs=[
            pl.BlockSpec((BLK, D), lambda i: (i, 0)),
            pl.BlockSpec((BLK, 1), lambda i: (i, 0)),
        ],
        out_shape=[
            jax.ShapeDtypeStruct((NPAD, D), jnp.float32),
            jax.ShapeDtypeStruct((NPAD, 1), jnp.float32),
        ],
    )(h1, degp)


def _tc_mid(part1, g1, dis, b1r, W2):
    return pl.pallas_call(
        _tc_mid_body,
        grid=_GRID,
        in_specs=[
            pl.BlockSpec((NC, BLK, D), lambda i: (0, i, 0)),
            pl.BlockSpec((BLK, D), lambda i: (i, 0)),
            pl.BlockSpec((BLK, 1), lambda i: (i, 0)),
            pl.BlockSpec((1, D), lambda i: (0, 0)),
            pl.BlockSpec((D, D), lambda i: (0, 0)),
        ],
        out_specs=pl.BlockSpec((BLK, D), lambda i: (i, 0)),
        out_shape=jax.ShapeDtypeStruct((NPAD, D), jnp.float32),
    )(part1, g1, dis, b1r, W2)


def _tc_out(part2, g2, dis, b2r):
    return pl.pallas_call(
        _tc_out_body,
        grid=_GRID,
        in_specs=[
            pl.BlockSpec((NC, BLK, D), lambda i: (0, i, 0)),
            pl.BlockSpec((BLK, D), lambda i: (i, 0)),
            pl.BlockSpec((BLK, 1), lambda i: (i, 0)),
            pl.BlockSpec((1, D), lambda i: (0, 0)),
        ],
        out_specs=pl.BlockSpec((BLK, D), lambda i: (i, 0)),
        out_shape=jax.ShapeDtypeStruct((N, D), jnp.float32),
    )(part2, g2, dis, b2r)


def kernel(x, edge_index, W1, b1, W2, b2):
    pad_e = E_PAD - E
    pad_i = np.arange(pad_e, dtype=np.int64)
    pad_src = jnp.asarray((pad_i * 37) % N, dtype=jnp.int32)
    pad_dst = jnp.asarray(N + pad_i % (NPAD - N), dtype=jnp.int32)
    src_p = jnp.concatenate([edge_index[0], pad_src]).reshape(TILES, CHUNKS, CW)
    dst_p = jnp.concatenate([edge_index[1], pad_dst]).reshape(TILES, CHUNKS, CW)
    x_p = jnp.concatenate([x, jnp.zeros((NPAD - N, D), x.dtype)], axis=0)
    zeros128 = jnp.zeros((NPAD, D), jnp.float32)
    ones128 = jnp.ones((CW, D), jnp.float32)
    b1r = b1.reshape(1, D)
    b2r = b2.reshape(1, D)

    degp = _sc_degree_kernel()(dst_p, zeros128, ones128)
    h1 = _tc_mm(x_p, W1)
    g1, dis = _tc_scale(h1, degp)
    part1 = _sc_agg_kernel()(g1, src_p, dst_p, zeros128)
    g2 = _tc_mid(part1, g1, dis, b1r, W2)
    part2 = _sc_agg_kernel()(g2, src_p, dst_p, zeros128)
    return _tc_out(part2, g2, dis, b2r)


# speedup vs baseline: 1.0232x; 1.0044x over previous
import functools

import numpy as np

import jax
import jax.numpy as jnp
from jax import lax
from jax.experimental import pallas as pl
from jax.experimental.pallas import tpu as pltpu
from jax.experimental.pallas import tpu_sc as plsc

N = 10000
E = 320000
D = 128

NC, NS = 2, 16
TILES = NC * NS
NPAD = 10240
CW = 128
EPT = NPAD
CHUNKS = EPT // CW
E_PAD = TILES * EPT
STRIPE = NPAD // NS
BLK = 1024


def _mesh():
    return plsc.VectorSubcoreMesh(
        core_axis_name="c", subcore_axis_name="s", num_cores=NC, num_subcores=NS
    )


def _deg_body(dst_r, zeros, ones, out, idx_v, obuf, deg_sh,
              semd0, semd1, semd2, semd3):
    c = lax.axis_index("c")
    s = lax.axis_index("s")
    wid = s * NC + c
    stripe = s * STRIPE
    pltpu.sync_copy(zeros.at[pl.ds(stripe, STRIPE)],
                    deg_sh.at[pl.ds(stripe, STRIPE)])
    pltpu.sync_copy(ones, obuf)
    pltpu.sync_copy(dst_r.at[wid], idx_v)
    plsc.subcore_barrier()

    semd = (semd0, semd1, semd2, semd3)

    def body(jj, carry):
        for b in range(4):
            j = jj * 4 + b

            @pl.when(j >= 4)
            def _():
                pltpu.make_async_copy(
                    obuf, deg_sh.at[idx_v.at[j]], semd[b]
                ).wait()

            pltpu.async_copy(obuf, deg_sh.at[idx_v.at[j]], semd[b], add=True)
        return carry

    lax.fori_loop(0, CHUNKS // 4, body, 0)
    for b in range(4):
        pltpu.make_async_copy(obuf, deg_sh.at[idx_v.at[0]], semd[b]).wait()
    plsc.subcore_barrier()
    pltpu.sync_copy(deg_sh.at[pl.ds(stripe, STRIPE)],
                    out.at[c, pl.ds(stripe, STRIPE)])


@functools.cache
def _sc_degree_kernel():
    return pl.kernel(
        _deg_body,
        out_type=jax.ShapeDtypeStruct((NC, NPAD, D), jnp.float32),
        mesh=_mesh(),
        scratch_types=[
            pltpu.VMEM((CHUNKS, CW), jnp.int32),
            pltpu.VMEM((CW, D), jnp.float32),
            pltpu.VMEM_SHARED((NPAD, D), jnp.float32),
            pltpu.SemaphoreType.DMA,
            pltpu.SemaphoreType.DMA,
            pltpu.SemaphoreType.DMA,
            pltpu.SemaphoreType.DMA,
        ],
    )


G = 8
NGRP = CHUNKS // G


def _agg_body(g, src_r, dst_r, zeros, part, sr, dst_v, gbuf, acc_sh,
              semg0, semg1, sems0, sems1, semi):
    c = lax.axis_index("c")
    s = lax.axis_index("s")
    wid = s * NC + c
    stripe = s * STRIPE
    pltpu.sync_copy(zeros.at[pl.ds(stripe, STRIPE)],
                    acc_sh.at[pl.ds(stripe, STRIPE)])
    pltpu.sync_copy(dst_r.at[wid], dst_v)
    pltpu.sync_copy(src_r.at[wid, pl.ds(0, G)], sr.at[0])
    pltpu.async_copy(src_r.at[wid, pl.ds(G, G)], sr.at[1], semi)
    plsc.subcore_barrier()

    semg = (semg0, semg1)
    sems = (sems0, sems1)
    pltpu.async_copy(g.at[sr.at[0, 0]], gbuf.at[0], semg0)

    def body(jj, carry):
        for b in range(2):
            bb = 1 - b
            j = jj * 2 + b

            @pl.when(jnp.logical_and(j % G == G - 1, j // G + 1 < NGRP))
            def _():
                pltpu.make_async_copy(
                    src_r.at[wid, pl.ds(0, G)], sr.at[0], semi
                ).wait()

            @pl.when(j >= 1)
            def _():
                pltpu.make_async_copy(
                    gbuf.at[bb], acc_sh.at[dst_v.at[j]], sems[bb]
                ).wait()

            @pl.when(j + 1 < CHUNKS)
            def _():
                jn = j + 1
                pltpu.async_copy(
                    g.at[sr.at[(jn // G) % 2, jn % G]], gbuf.at[bb], semg[bb]
                )

            pltpu.make_async_copy(
                g.at[sr.at[0, 0]], gbuf.at[b], semg[b]
            ).wait()
            pltpu.async_copy(gbuf.at[b], acc_sh.at[dst_v.at[j]], sems[b],
                             add=True)

            @pl.when(jnp.logical_and(j % G == G - 1, j // G + 2 < NGRP))
            def _():
                grp = j // G
                pltpu.async_copy(
                    src_r.at[wid, pl.ds((grp + 2) * G, G)], sr.at[grp % 2],
                    semi,
                )
        return carry

    lax.fori_loop(0, CHUNKS // 2, body, 0)
    pltpu.make_async_copy(gbuf.at[1], acc_sh.at[dst_v.at[0]], sems1).wait()
    plsc.subcore_barrier()
    pltpu.sync_copy(acc_sh.at[pl.ds(stripe, STRIPE)],
                    part.at[c, pl.ds(stripe, STRIPE)])


@functools.cache
def _sc_agg_kernel():
    return pl.kernel(
        _agg_body,
        out_type=jax.ShapeDtypeStruct((NC, NPAD, D), jnp.float32),
        mesh=_mesh(),
        scratch_types=[
            pltpu.VMEM((2, G, CW), jnp.int32),
            pltpu.VMEM((CHUNKS, CW), jnp.int32),
            pltpu.VMEM((2, CW, D), jnp.float32),
            pltpu.VMEM_SHARED((NPAD, D), jnp.float32),
            pltpu.SemaphoreType.DMA,
            pltpu.SemaphoreType.DMA,
            pltpu.SemaphoreType.DMA,
            pltpu.SemaphoreType.DMA,
            pltpu.SemaphoreType.DMA,
        ],
    )


def _tc_mm_body(x_ref, w_ref, h_ref):
    h_ref[...] = jnp.dot(
        x_ref[...], w_ref[...], preferred_element_type=jnp.float32
    )


def _tc_scale_body(h_ref, deg_ref, g_ref, dis_ref):
    deg = deg_ref[0][:, 0:1] + deg_ref[1][:, 0:1] + 1.0
    dis = lax.rsqrt(deg)
    g_ref[...] = h_ref[...] * dis
    dis_ref[...] = dis


def _tc_mid_body(p_ref, g1_ref, dis_ref, b1_ref, w2_ref, g2_ref):
    dis = dis_ref[...]
    a = (p_ref[0] + p_ref[1] + g1_ref[...]) * dis + b1_ref[...]
    h = jnp.maximum(a, 0.0)
    g2_ref[...] = (
        jnp.dot(h, w2_ref[...], preferred_element_type=jnp.float32) * dis
    )


def _tc_out_body(p_ref, g2_ref, dis_ref, b2_ref, o_ref):
    z = (p_ref[0] + p_ref[1] + g2_ref[...]) * dis_ref[...] + b2_ref[...]
    m = jnp.max(z, axis=1, keepdims=True)
    sh = z - m
    o_ref[...] = sh - jnp.log(jnp.sum(jnp.exp(sh), axis=1, keepdims=True))


_GRID = (NPAD // BLK,)


def _tc_mm(x_p, W1):
    return pl.pallas_call(
        _tc_mm_body,
        grid=_GRID,
        in_specs=[
            pl.BlockSpec((BLK, D), lambda i: (i, 0)),
            pl.BlockSpec((D, D), lambda i: (0, 0)),
        ],
        out_specs=pl.BlockSpec((BLK, D), lambda i: (i, 0)),
        out_shape=jax.ShapeDtypeStruct((NPAD, D), jnp.float32),
    )(x_p, W1)


def _tc_scale(h1, degp):
    return pl.pallas_call(
        _tc_scale_body,
        grid=_GRID,
        in_specs=[
            pl.BlockSpec((BLK, D), lambda i: (i, 0)),
            pl.BlockSpec((NC, BLK, D), lambda i: (0, i, 0)),
        ],
        out_specs=[
            pl.BlockSpec((BLK, D), lambda i: (i, 0)),
            pl.BlockSpec((BLK, 1), lambda i: (i, 0)),
        ],
        out_shape=[
            jax.ShapeDtypeStruct((NPAD, D), jnp.float32),
            jax.ShapeDtypeStruct((NPAD, 1), jnp.float32),
        ],
    )(h1, degp)


def _tc_mid(part1, g1, dis, b1r, W2):
    return pl.pallas_call(
        _tc_mid_body,
        grid=_GRID,
        in_specs=[
            pl.BlockSpec((NC, BLK, D), lambda i: (0, i, 0)),
            pl.BlockSpec((BLK, D), lambda i: (i, 0)),
            pl.BlockSpec((BLK, 1), lambda i: (i, 0)),
            pl.BlockSpec((1, D), lambda i: (0, 0)),
            pl.BlockSpec((D, D), lambda i: (0, 0)),
        ],
        out_specs=pl.BlockSpec((BLK, D), lambda i: (i, 0)),
        out_shape=jax.ShapeDtypeStruct((NPAD, D), jnp.float32),
    )(part1, g1, dis, b1r, W2)


def _tc_out(part2, g2, dis, b2r):
    return pl.pallas_call(
        _tc_out_body,
        grid=_GRID,
        in_specs=[
            pl.BlockSpec((NC, BLK, D), lambda i: (0, i, 0)),
            pl.BlockSpec((BLK, D), lambda i: (i, 0)),
            pl.BlockSpec((BLK, 1), lambda i: (i, 0)),
            pl.BlockSpec((1, D), lambda i: (0, 0)),
        ],
        out_specs=pl.BlockSpec((BLK, D), lambda i: (i, 0)),
        out_shape=jax.ShapeDtypeStruct((N, D), jnp.float32),
    )(part2, g2, dis, b2r)


def kernel(x, edge_index, W1, b1, W2, b2):
    pad_e = E_PAD - E
    pad_i = np.arange(pad_e, dtype=np.int64)
    pad_src = jnp.asarray((pad_i * 37) % N, dtype=jnp.int32)
    pad_dst = jnp.asarray(N + pad_i % (NPAD - N), dtype=jnp.int32)
    src_p = jnp.concatenate([edge_index[0], pad_src]).reshape(TILES, CHUNKS, CW)
    dst_p = jnp.concatenate([edge_index[1], pad_dst]).reshape(TILES, CHUNKS, CW)
    x_p = jnp.concatenate([x, jnp.zeros((NPAD - N, D), x.dtype)], axis=0)
    zeros128 = jnp.zeros((NPAD, D), jnp.float32)
    ones128 = jnp.ones((CW, D), jnp.float32)
    b1r = b1.reshape(1, D)
    b2r = b2.reshape(1, D)

    degp = _sc_degree_kernel()(dst_p, zeros128, ones128)
    h1 = _tc_mm(x_p, W1)
    g1, dis = _tc_scale(h1, degp)
    part1 = _sc_agg_kernel()(g1, src_p, dst_p, zeros128)
    g2 = _tc_mid(part1, g1, dis, b1r, W2)
    part2 = _sc_agg_kernel()(g2, src_p, dst_p, zeros128)
    return _tc_out(part2, g2, dis, b2r)
